# baseline (speedup 1.0000x reference)
import jax
import jax.numpy as jnp
from jax.experimental import pallas as pl
from jax.experimental.pallas import tpu as pltpu
from jax.experimental.pallas import tpu_sc as plsc


_NUM_WORKERS = 32


def _sc_gather(user_emb, movie_emb, uids, mids):
    B = uids.shape[0]
    D = user_emb.shape[1]
    per_w = B // _NUM_WORKERS
    half = per_w // 2
    mesh = plsc.VectorSubcoreMesh(core_axis_name="core", subcore_axis_name="subcore")

    n_chunks = 4
    chunk = per_w // n_chunks
    n_bufs = 4

    @pl.kernel(
        out_type=(
            jax.ShapeDtypeStruct((B, D), user_emb.dtype),
            jax.ShapeDtypeStruct((B, D), movie_emb.dtype),
        ),
        mesh=mesh,
        scratch_types=[
            pltpu.VMEM((half,), jnp.int32),
            pltpu.VMEM((half,), jnp.int32),
            pltpu.VMEM((half,), jnp.int32),
            pltpu.VMEM((half,), jnp.int32),
            pltpu.VMEM((chunk, D), jnp.float32),
            pltpu.VMEM((chunk, D), jnp.float32),
            pltpu.VMEM((chunk, D), jnp.float32),
            pltpu.VMEM((chunk, D), jnp.float32),
            pltpu.SemaphoreType.DMA,
            pltpu.SemaphoreType.DMA,
            pltpu.SemaphoreType.DMA,
            pltpu.SemaphoreType.DMA,
            pltpu.SemaphoreType.DMA,
        ],
    )
    def gather_kernel(ue_hbm, me_hbm, ui_hbm, mi_hbm, ou_hbm, om_hbm,
                      iu0, iu1, im0, im1, buf0, buf1, buf2, buf3,
                      sem0, sem1, sem2, sem3, sem_idx):
        wid = (jax.lax.axis_index("subcore") * 2 + jax.lax.axis_index("core"))
        base = wid * per_w
        idx_cps = [
            pltpu.async_copy(ui_hbm.at[pl.ds(base, half)], iu0, sem_idx),
            pltpu.async_copy(ui_hbm.at[pl.ds(base + half, half)], iu1, sem_idx),
            pltpu.async_copy(mi_hbm.at[pl.ds(base, half)], im0, sem_idx),
            pltpu.async_copy(mi_hbm.at[pl.ds(base + half, half)], im1, sem_idx),
        ]
        for cp in idx_cps:
            cp.wait()
        items = []
        for t_hbm, ia, ib, o_hbm in ((ue_hbm, iu0, iu1, ou_hbm),
                                     (me_hbm, im0, im1, om_hbm)):
            for j in range(n_chunks):
                iref = ia if j < 2 else ib
                ioff = (j % 2) * chunk
                items.append((t_hbm, iref, ioff, o_hbm, j * chunk))
        bufs = (buf0, buf1, buf2, buf3)
        sems = (sem0, sem1, sem2, sem3)
        n = len(items)
        pending = [None] * n

        def start(i):
            t_hbm, iref, ioff, _, _ = items[i]
            pending[i] = pltpu.async_copy(
                t_hbm.at[iref.at[pl.ds(ioff, chunk)]],
                bufs[i % n_bufs], sems[i % n_bufs])

        for i in range(n_bufs - 1):
            start(i)
        for i in range(n):
            pending[i].wait()
            _, _, _, out, off = items[i]
            if i + n_bufs - 1 < n:
                start(i + n_bufs - 1)
            pltpu.sync_copy(bufs[i % n_bufs], out.at[pl.ds(base + off, chunk)])

    return gather_kernel(user_emb, movie_emb, uids, mids)


_MLP_BLOCK = 8192


def _mlp_body(u_ref, m_ref, w1_ref, b1_ref, w2_ref, b2_ref,
              w3_ref, b3_ref, o_ref):
    D = u_ref.shape[1]
    h = jax.lax.dot_general(u_ref[...], w1_ref[:, :D], (((1,), (1,)), ((), ())),
                            preferred_element_type=jnp.float32)
    h = h + jax.lax.dot_general(m_ref[...], w1_ref[:, D:], (((1,), (1,)), ((), ())),
                                preferred_element_type=jnp.float32)
    h = jnp.maximum(h + b1_ref[...].reshape(1, -1), 0.0)
    h2t = jax.lax.dot_general(w2_ref[...], h, (((1,), (1,)), ((), ())),
                              preferred_element_type=jnp.float32)
    h2t = jnp.maximum(h2t + b2_ref[...].reshape(-1, 1), 0.0)
    ot = jax.lax.dot_general(w3_ref[...], h2t, (((1,), (0,)), ((), ())),
                             preferred_element_type=jnp.float32)
    o_ref[...] = ot + b3_ref[0]


def _mlp(u, m, W1, b1, W2, b2, W3, b3):
    B, D = u.shape
    grid = (B // _MLP_BLOCK,)
    out_t = pl.pallas_call(
        _mlp_body,
        grid=grid,
        in_specs=[
            pl.BlockSpec((_MLP_BLOCK, D), lambda i: (i, 0)),
            pl.BlockSpec((_MLP_BLOCK, D), lambda i: (i, 0)),
            pl.BlockSpec(W1.shape, lambda i: (0, 0)),
            pl.BlockSpec(b1.shape, lambda i: (0,)),
            pl.BlockSpec(W2.shape, lambda i: (0, 0)),
            pl.BlockSpec(b2.shape, lambda i: (0,)),
            pl.BlockSpec(W3.shape, lambda i: (0, 0)),
            pl.BlockSpec(b3.shape, lambda i: (0,)),
        ],
        out_specs=pl.BlockSpec((1, _MLP_BLOCK), lambda i: (0, i)),
        out_shape=jax.ShapeDtypeStruct((1, B), jnp.float32),
        compiler_params=pltpu.CompilerParams(
            dimension_semantics=("parallel",),
        ),
    )(u, m, W1, b1, W2, b2, W3, b3)
    return out_t.reshape(B)


def kernel(user_ids, movie_ids, user_emb, movie_emb, W1, b1, W2, b2, W3, b3):
    u, m = _sc_gather(user_emb, movie_emb, user_ids, movie_ids)
    return _mlp(u, m, W1, b1, W2, b2, W3, b3)


# speedup vs baseline: 1.0259x; 1.0259x over previous
import jax
import jax.numpy as jnp
from jax.experimental import pallas as pl
from jax.experimental.pallas import tpu as pltpu
from jax.experimental.pallas import tpu_sc as plsc


_NUM_WORKERS = 32


def _sc_gather(user_emb, movie_emb, uids, mids):
    B = uids.shape[0]
    D = user_emb.shape[1]
    per_w = B // _NUM_WORKERS
    half = per_w // 2
    mesh = plsc.VectorSubcoreMesh(core_axis_name="core", subcore_axis_name="subcore")

    n_chunks = 4
    chunk = per_w // n_chunks
    n_bufs = 4

    @pl.kernel(
        out_type=jax.ShapeDtypeStruct((B, 2 * D), user_emb.dtype),
        mesh=mesh,
        scratch_types=[
            pltpu.VMEM((half,), jnp.int32),
            pltpu.VMEM((half,), jnp.int32),
            pltpu.VMEM((half,), jnp.int32),
            pltpu.VMEM((half,), jnp.int32),
            pltpu.VMEM((chunk, D), jnp.float32),
            pltpu.VMEM((chunk, D), jnp.float32),
            pltpu.VMEM((chunk, D), jnp.float32),
            pltpu.VMEM((chunk, D), jnp.float32),
            pltpu.SemaphoreType.DMA,
            pltpu.SemaphoreType.DMA,
            pltpu.SemaphoreType.DMA,
            pltpu.SemaphoreType.DMA,
            pltpu.SemaphoreType.DMA,
        ],
    )
    def gather_kernel(ue_hbm, me_hbm, ui_hbm, mi_hbm, ox_hbm,
                      iu0, iu1, im0, im1, buf0, buf1, buf2, buf3,
                      sem0, sem1, sem2, sem3, sem_idx):
        wid = (jax.lax.axis_index("subcore") * 2 + jax.lax.axis_index("core"))
        base = wid * per_w
        idx_cps = [
            pltpu.async_copy(ui_hbm.at[pl.ds(base, half)], iu0, sem_idx),
            pltpu.async_copy(ui_hbm.at[pl.ds(base + half, half)], iu1, sem_idx),
            pltpu.async_copy(mi_hbm.at[pl.ds(base, half)], im0, sem_idx),
            pltpu.async_copy(mi_hbm.at[pl.ds(base + half, half)], im1, sem_idx),
        ]
        for cp in idx_cps:
            cp.wait()
        items = []
        for t_hbm, ia, ib, col in ((ue_hbm, iu0, iu1, 0),
                                   (me_hbm, im0, im1, D)):
            for j in range(n_chunks):
                iref = ia if j < 2 else ib
                ioff = (j % 2) * chunk
                items.append((t_hbm, iref, ioff, col, j * chunk))
        bufs = (buf0, buf1, buf2, buf3)
        sems = (sem0, sem1, sem2, sem3)
        n = len(items)
        pending = [None] * n

        def start(i):
            t_hbm, iref, ioff, _, _ = items[i]
            pending[i] = pltpu.async_copy(
                t_hbm.at[iref.at[pl.ds(ioff, chunk)]],
                bufs[i % n_bufs], sems[i % n_bufs])

        for i in range(n_bufs - 1):
            start(i)
        for i in range(n):
            pending[i].wait()
            _, _, _, col, off = items[i]
            if i + n_bufs - 1 < n:
                start(i + n_bufs - 1)
            pltpu.sync_copy(bufs[i % n_bufs],
                            ox_hbm.at[pl.ds(base + off, chunk), pl.ds(col, D)])

    return gather_kernel(user_emb, movie_emb, uids, mids)


_MLP_BLOCK = 8192


def _mlp_body(x_ref, w1_ref, b1_ref, w2_ref, b2_ref,
              w3_ref, b3_ref, o_ref):
    h = jax.lax.dot_general(x_ref[...], w1_ref[...], (((1,), (1,)), ((), ())),
                            preferred_element_type=jnp.float32)
    h = jnp.maximum(h + b1_ref[...].reshape(1, -1), 0.0)
    h2t = jax.lax.dot_general(w2_ref[...], h, (((1,), (1,)), ((), ())),
                              preferred_element_type=jnp.float32)
    h2t = jnp.maximum(h2t + b2_ref[...].reshape(-1, 1), 0.0)
    ot = jax.lax.dot_general(w3_ref[...], h2t, (((1,), (0,)), ((), ())),
                             preferred_element_type=jnp.float32)
    o_ref[...] = ot + b3_ref[0]


def _mlp(x, W1, b1, W2, b2, W3, b3):
    B, D2 = x.shape
    grid = (B // _MLP_BLOCK,)
    out_t = pl.pallas_call(
        _mlp_body,
        grid=grid,
        in_specs=[
            pl.BlockSpec((_MLP_BLOCK, D2), lambda i: (i, 0)),
            pl.BlockSpec(W1.shape, lambda i: (0, 0)),
            pl.BlockSpec(b1.shape, lambda i: (0,)),
            pl.BlockSpec(W2.shape, lambda i: (0, 0)),
            pl.BlockSpec(b2.shape, lambda i: (0,)),
            pl.BlockSpec(W3.shape, lambda i: (0, 0)),
            pl.BlockSpec(b3.shape, lambda i: (0,)),
        ],
        out_specs=pl.BlockSpec((1, _MLP_BLOCK), lambda i: (0, i)),
        out_shape=jax.ShapeDtypeStruct((1, B), jnp.float32),
        compiler_params=pltpu.CompilerParams(
            dimension_semantics=("parallel",),
        ),
    )(x, W1, b1, W2, b2, W3, b3)
    return out_t.reshape(B)


def kernel(user_ids, movie_ids, user_emb, movie_emb, W1, b1, W2, b2, W3, b3):
    x = _sc_gather(user_emb, movie_emb, user_ids, movie_ids)
    return _mlp(x, W1, b1, W2, b2, W3, b3)
